# baseline (speedup 1.0000x reference)
import functools

import jax
import jax.numpy as jnp
from jax import lax
from jax.experimental import pallas as pl
from jax.experimental.pallas import tpu as pltpu
from jax.experimental.pallas import tpu_sc as plsc

V = 100000
D = 64
NB = 781
TAIL0 = NB * 128
TAILW = V - TAIL0
P = V // 2
NC = 2
NS = 16
NW = NC * NS
NBIG = NB - NW * (NB // NW)
CP = pltpu.CompilerParams(use_tc_tiling_on_sc=True, needs_layout_passes=False)


def _worker_range(wid):
    n = jnp.where(wid < NBIG, NB // NW + 1, NB // NW)
    start = jnp.where(
        wid < NBIG, (NB // NW + 1) * wid, NB // NW * wid + NBIG
    )
    return start, n


def _iota16():
    return lax.iota(jnp.int32, 16)


@functools.cache
def _ka():
    mesh = plsc.VectorSubcoreMesh(core_axis_name="c", subcore_axis_name="s")

    @functools.partial(
        pl.kernel,
        mesh=mesh,
        out_type=jax.ShapeDtypeStruct((P, 128), jnp.float32),
        scratch_types=[
            pltpu.VMEM((D, 128), jnp.float32),
            pltpu.VMEM((D, 128), jnp.float32),
            pltpu.VMEM((D, 128), jnp.float32),
            pltpu.VMEM((D, 128), jnp.float32),
            pltpu.VMEM((D, TAILW), jnp.float32),
            pltpu.VMEM((TAILW // 2, 128), jnp.float32),
            pltpu.SemaphoreType.DMA,
            pltpu.SemaphoreType.DMA,
            pltpu.SemaphoreType.DMA,
            pltpu.SemaphoreType.DMA,
        ],
        compiler_params=CP,
    )
    def transpose_kernel(
        wt_hbm, pairs_hbm, a0, a1, b0, b1, atail, btail, g0, g1, w0, w1
    ):
        wid = lax.axis_index("s") * NC + lax.axis_index("c")
        start, n = _worker_range(wid)
        ablk = (a0, a1)
        bblk = (b0, b1)
        gsem = (g0, g1)
        wsem = (w0, w1)
        rows4 = [_iota16() + 16 * t for t in range(4)]

        def in_dma(k, par):
            return pltpu.async_copy(
                wt_hbm.at[:, pl.ds((start + k) * 128, 128)], ablk[par], gsem[par]
            )

        def out_dma(k, par):
            return pltpu.async_copy(
                bblk[par], pairs_hbm.at[pl.ds((start + k) * 64, 64)], wsem[par]
            )

        def permute(src, dst, q):
            ce = jnp.full((16,), 2 * q, jnp.int32)
            co = ce + 1
            for t in range(4):
                dst[q, pl.ds(16 * t, 16)] = plsc.load_gather(src, [rows4[t], ce])
                dst[q, pl.ds(64 + 16 * t, 16)] = plsc.load_gather(
                    src, [rows4[t], co]
                )

        in_dma(0, 0)

        @pl.when(n > 1)
        def _():
            in_dma(1, 1)

        def body(i, carry):
            for par in range(2):
                k = 2 * i + par

                @pl.when(k < n)
                def _():
                    pltpu.make_async_copy(
                        wt_hbm.at[:, pl.ds(0, 128)], ablk[par], gsem[par]
                    ).wait()

                    @pl.when(k >= 2)
                    def _():
                        pltpu.make_async_copy(
                            bblk[par], pairs_hbm.at[pl.ds(0, 64)], wsem[par]
                        ).wait()

                    def qbody(q, c):
                        permute(ablk[par], bblk[par], q)
                        return c

                    lax.fori_loop(0, D, qbody, 0)
                    out_dma(k, par)

                    @pl.when(k + 2 < n)
                    def _():
                        in_dma(k + 2, par)

            return carry

        lax.fori_loop(0, 13, body, 0)
        for par in range(2):

            @pl.when(n > par)
            def _():
                pltpu.make_async_copy(
                    bblk[par], pairs_hbm.at[pl.ds(0, 64)], wsem[par]
                ).wait()

        @pl.when(wid == NW - 1)
        def _():
            pltpu.sync_copy(wt_hbm.at[:, pl.ds(TAIL0, TAILW)], atail)

            def qbody(q, c):
                ce = jnp.full((16,), 2 * q, jnp.int32)
                co = ce + 1
                for t in range(4):
                    btail[q, pl.ds(16 * t, 16)] = plsc.load_gather(
                        atail, [rows4[t], ce]
                    )
                    btail[q, pl.ds(64 + 16 * t, 16)] = plsc.load_gather(
                        atail, [rows4[t], co]
                    )
                return c

            lax.fori_loop(0, TAILW // 2, qbody, 0)
            pltpu.sync_copy(btail, pairs_hbm.at[pl.ds(P - TAILW // 2, TAILW // 2)])

    return transpose_kernel


@functools.cache
def _kb():
    mesh = plsc.VectorSubcoreMesh(core_axis_name="c", subcore_axis_name="s")

    @functools.partial(
        pl.kernel,
        mesh=mesh,
        out_type=jax.ShapeDtypeStruct((D, V), jnp.float32),
        scratch_types=[
            pltpu.VMEM((3200,), jnp.int32),
            pltpu.VMEM((128,), jnp.int32),
            pltpu.VMEM((128,), jnp.int32),
            pltpu.VMEM((128, 128), jnp.float32),
            pltpu.VMEM((128, 128), jnp.float32),
            pltpu.VMEM((D, 128), jnp.float32),
            pltpu.VMEM((D, 128), jnp.float32),
            pltpu.VMEM((TAILW,), jnp.int32),
            pltpu.VMEM((TAILW,), jnp.int32),
            pltpu.VMEM((TAILW, 128), jnp.float32),
            pltpu.VMEM((D, TAILW), jnp.float32),
            pltpu.SemaphoreType.DMA,
            pltpu.SemaphoreType.DMA,
            pltpu.SemaphoreType.DMA,
            pltpu.SemaphoreType.DMA,
            pltpu.SemaphoreType.DMA,
        ],
        compiler_params=CP,
    )
    def gather_kernel(
        pairs_hbm,
        idx_hbm,
        out_hbm,
        cidx,
        p0,
        p1,
        gb0,
        gb1,
        ob0,
        ob1,
        cidxt,
        pidxt,
        gtail,
        otail,
        g0,
        g1,
        w0,
        w1,
        tsem,
    ):
        wid = lax.axis_index("s") * NC + lax.axis_index("c")
        start, n = _worker_range(wid)
        bufst = jnp.minimum(start, NB - 25)
        lo = start - bufst
        pidx = (p0, p1)
        gblk = (gb0, gb1)
        oblk = (ob0, ob1)
        gsem = (g0, g1)
        wsem = (w0, w1)
        rows8 = [_iota16() + 16 * g for g in range(8)]

        pltpu.sync_copy(idx_hbm.at[pl.ds(bufst * 128, 3200)], cidx)

        def compute_pidx_and_fire(k, par):
            base = (lo + k) * 128
            for g in range(8):
                pidx[par][pl.ds(16 * g, 16)] = (
                    cidx[pl.ds(base + 16 * g, 16)] >> 1
                )
            pltpu.async_copy(pairs_hbm.at[pidx[par]], gblk[par], gsem[par])

        compute_pidx_and_fire(0, 0)

        @pl.when(n > 1)
        def _():
            compute_pidx_and_fire(1, 1)

        def body(i, carry):
            for par in range(2):
                k = 2 * i + par

                @pl.when(k < n)
                def _():
                    pltpu.make_async_copy(
                        pairs_hbm.at[pidx[par]], gblk[par], gsem[par]
                    ).wait()

                    @pl.when(k >= 2)
                    def _():
                        pltpu.make_async_copy(
                            oblk[par], out_hbm.at[:, pl.ds(0, 128)], wsem[par]
                        ).wait()

                    base = (lo + k) * 128
                    h64 = tuple(
                        (cidx[pl.ds(base + 16 * g, 16)] & 1) << 6
                        for g in range(8)
                    )

                    def fbody(f, c):
                        for g in range(8):
                            ob = oblk[par]
                            ob[f, pl.ds(16 * g, 16)] = plsc.load_gather(
                                gblk[par], [rows8[g], c[g] + f]
                            )
                        return c

                    lax.fori_loop(0, D, fbody, h64)
                    pltpu.async_copy(
                        oblk[par],
                        out_hbm.at[:, pl.ds((start + k) * 128, 128)],
                        wsem[par],
                    )

                    @pl.when(k + 2 < n)
                    def _():
                        compute_pidx_and_fire(k + 2, par)

            return carry

        lax.fori_loop(0, 13, body, 0)
        for par in range(2):

            @pl.when(n > par)
            def _():
                pltpu.make_async_copy(
                    oblk[par], out_hbm.at[:, pl.ds(0, 128)], wsem[par]
                ).wait()

        @pl.when(wid == NW - 1)
        def _():
            pltpu.sync_copy(idx_hbm.at[pl.ds(TAIL0, TAILW)], cidxt)
            for g in range(2):
                pidxt[pl.ds(16 * g, 16)] = cidxt[pl.ds(16 * g, 16)] >> 1
            pltpu.async_copy(pairs_hbm.at[pidxt], gtail, tsem).wait()
            h64t = tuple(
                (cidxt[pl.ds(16 * g, 16)] & 1) << 6 for g in range(2)
            )

            def fbody(f, c):
                for g in range(2):
                    otail[f, pl.ds(16 * g, 16)] = plsc.load_gather(
                        gtail, [rows8[g], c[g] + f]
                    )
                return c

            lax.fori_loop(0, D, fbody, h64t)
            pltpu.sync_copy(otail, out_hbm.at[:, pl.ds(TAIL0, TAILW)])

    return gather_kernel


def kernel(x, emb_weight):
    idx = x[:, 0].astype(jnp.int32)
    wt = emb_weight.T
    pairs = _ka()(wt)
    out_t = _kb()(pairs, idx)
    return out_t.T


# speedup vs baseline: 1.7419x; 1.7419x over previous
import functools

import jax
import jax.numpy as jnp
from jax import lax
from jax.experimental import pallas as pl
from jax.experimental.pallas import tpu as pltpu
from jax.experimental.pallas import tpu_sc as plsc

V = 100000
D = 64
NB = 781
TAIL0 = NB * 128
TAILW = V - TAIL0
P = V // 2
NC = 2
NS = 16
NW = NC * NS
NBIG = NB - NW * (NB // NW)
CP = pltpu.CompilerParams(use_tc_tiling_on_sc=True, needs_layout_passes=False)


def _worker_range(wid):
    n = jnp.where(wid < NBIG, NB // NW + 1, NB // NW)
    start = jnp.where(
        wid < NBIG, (NB // NW + 1) * wid, NB // NW * wid + NBIG
    )
    return start, n


def _iota16():
    return lax.iota(jnp.int32, 16)


@functools.cache
def _ka():
    mesh = plsc.VectorSubcoreMesh(core_axis_name="c", subcore_axis_name="s")

    @functools.partial(
        pl.kernel,
        mesh=mesh,
        out_type=jax.ShapeDtypeStruct((P, 128), jnp.float32),
        scratch_types=[
            pltpu.VMEM((D, 128), jnp.float32),
            pltpu.VMEM((D, 128), jnp.float32),
            pltpu.VMEM((D, 128), jnp.float32),
            pltpu.VMEM((D, 128), jnp.float32),
            pltpu.VMEM((D, TAILW), jnp.float32),
            pltpu.VMEM((TAILW // 2, 128), jnp.float32),
            pltpu.SemaphoreType.DMA,
            pltpu.SemaphoreType.DMA,
            pltpu.SemaphoreType.DMA,
            pltpu.SemaphoreType.DMA,
        ],
        compiler_params=CP,
    )
    def transpose_kernel(
        wt_hbm, pairs_hbm, a0, a1, b0, b1, atail, btail, g0, g1, w0, w1
    ):
        wid = lax.axis_index("s") * NC + lax.axis_index("c")
        start, n = _worker_range(wid)
        ablk = (a0, a1)
        bblk = (b0, b1)
        gsem = (g0, g1)
        wsem = (w0, w1)
        rows4 = [_iota16() + 16 * t for t in range(4)]

        def in_dma(k, par):
            return pltpu.async_copy(
                wt_hbm.at[:, pl.ds((start + k) * 128, 128)], ablk[par], gsem[par]
            )

        def out_dma(k, par):
            return pltpu.async_copy(
                bblk[par], pairs_hbm.at[pl.ds((start + k) * 64, 64)], wsem[par]
            )

        def permute(src, dst, q):
            ce = jnp.full((16,), 2 * q, jnp.int32)
            co = ce + 1
            for t in range(4):
                dst[q, pl.ds(16 * t, 16)] = plsc.load_gather(src, [rows4[t], ce])
                dst[q, pl.ds(64 + 16 * t, 16)] = plsc.load_gather(
                    src, [rows4[t], co]
                )

        in_dma(0, 0)

        @pl.when(n > 1)
        def _():
            in_dma(1, 1)

        def body(i, carry):
            for par in range(2):
                k = 2 * i + par

                @pl.when(k < n)
                def _():
                    pltpu.make_async_copy(
                        wt_hbm.at[:, pl.ds(0, 128)], ablk[par], gsem[par]
                    ).wait()

                    @pl.when(k >= 2)
                    def _():
                        pltpu.make_async_copy(
                            bblk[par], pairs_hbm.at[pl.ds(0, 64)], wsem[par]
                        ).wait()

                    @plsc.parallel_loop(0, D, unroll=4)
                    def _(q):
                        permute(ablk[par], bblk[par], q)

                    out_dma(k, par)

                    @pl.when(k + 2 < n)
                    def _():
                        in_dma(k + 2, par)

            return carry

        lax.fori_loop(0, 13, body, 0)
        for par in range(2):

            @pl.when(n > par)
            def _():
                pltpu.make_async_copy(
                    bblk[par], pairs_hbm.at[pl.ds(0, 64)], wsem[par]
                ).wait()

        @pl.when(wid == NW - 1)
        def _():
            pltpu.sync_copy(wt_hbm.at[:, pl.ds(TAIL0, TAILW)], atail)

            @plsc.parallel_loop(0, TAILW // 2, unroll=4)
            def _(q):
                ce = jnp.full((16,), 2 * q, jnp.int32)
                co = ce + 1
                for t in range(4):
                    btail[q, pl.ds(16 * t, 16)] = plsc.load_gather(
                        atail, [rows4[t], ce]
                    )
                    btail[q, pl.ds(64 + 16 * t, 16)] = plsc.load_gather(
                        atail, [rows4[t], co]
                    )
            pltpu.sync_copy(btail, pairs_hbm.at[pl.ds(P - TAILW // 2, TAILW // 2)])

    return transpose_kernel


@functools.cache
def _kb():
    mesh = plsc.VectorSubcoreMesh(core_axis_name="c", subcore_axis_name="s")

    @functools.partial(
        pl.kernel,
        mesh=mesh,
        out_type=jax.ShapeDtypeStruct((D, V), jnp.float32),
        scratch_types=[
            pltpu.VMEM((3200,), jnp.int32),
            pltpu.VMEM((128,), jnp.int32),
            pltpu.VMEM((128,), jnp.int32),
            pltpu.VMEM((128, 128), jnp.float32),
            pltpu.VMEM((128, 128), jnp.float32),
            pltpu.VMEM((D, 128), jnp.float32),
            pltpu.VMEM((D, 128), jnp.float32),
            pltpu.VMEM((TAILW,), jnp.int32),
            pltpu.VMEM((TAILW,), jnp.int32),
            pltpu.VMEM((TAILW, 128), jnp.float32),
            pltpu.VMEM((D, TAILW), jnp.float32),
            pltpu.SemaphoreType.DMA,
            pltpu.SemaphoreType.DMA,
            pltpu.SemaphoreType.DMA,
            pltpu.SemaphoreType.DMA,
            pltpu.SemaphoreType.DMA,
        ],
        compiler_params=CP,
    )
    def gather_kernel(
        pairs_hbm,
        idx_hbm,
        out_hbm,
        cidx,
        p0,
        p1,
        gb0,
        gb1,
        ob0,
        ob1,
        cidxt,
        pidxt,
        gtail,
        otail,
        g0,
        g1,
        w0,
        w1,
        tsem,
    ):
        wid = lax.axis_index("s") * NC + lax.axis_index("c")
        start, n = _worker_range(wid)
        bufst = jnp.minimum(start, NB - 25)
        lo = start - bufst
        pidx = (p0, p1)
        gblk = (gb0, gb1)
        oblk = (ob0, ob1)
        gsem = (g0, g1)
        wsem = (w0, w1)
        rows8 = [_iota16() + 16 * g for g in range(8)]

        pltpu.sync_copy(idx_hbm.at[pl.ds(bufst * 128, 3200)], cidx)

        def compute_pidx_and_fire(k, par):
            base = (lo + k) * 128
            for g in range(8):
                pidx[par][pl.ds(16 * g, 16)] = (
                    cidx[pl.ds(base + 16 * g, 16)] >> 1
                )
            pltpu.async_copy(pairs_hbm.at[pidx[par]], gblk[par], gsem[par])

        compute_pidx_and_fire(0, 0)

        @pl.when(n > 1)
        def _():
            compute_pidx_and_fire(1, 1)

        def body(i, carry):
            for par in range(2):
                k = 2 * i + par

                @pl.when(k < n)
                def _():
                    pltpu.make_async_copy(
                        pairs_hbm.at[pidx[par]], gblk[par], gsem[par]
                    ).wait()

                    @pl.when(k >= 2)
                    def _():
                        pltpu.make_async_copy(
                            oblk[par], out_hbm.at[:, pl.ds(0, 128)], wsem[par]
                        ).wait()

                    base = (lo + k) * 128
                    h64 = tuple(
                        (cidx[pl.ds(base + 16 * g, 16)] & 1) << 6
                        for g in range(8)
                    )

                    @plsc.parallel_loop(0, D, unroll=4)
                    def _(f):
                        for g in range(8):
                            ob = oblk[par]
                            ob[f, pl.ds(16 * g, 16)] = plsc.load_gather(
                                gblk[par], [rows8[g], h64[g] + f]
                            )
                    pltpu.async_copy(
                        oblk[par],
                        out_hbm.at[:, pl.ds((start + k) * 128, 128)],
                        wsem[par],
                    )

                    @pl.when(k + 2 < n)
                    def _():
                        compute_pidx_and_fire(k + 2, par)

            return carry

        lax.fori_loop(0, 13, body, 0)
        for par in range(2):

            @pl.when(n > par)
            def _():
                pltpu.make_async_copy(
                    oblk[par], out_hbm.at[:, pl.ds(0, 128)], wsem[par]
                ).wait()

        @pl.when(wid == NW - 1)
        def _():
            pltpu.sync_copy(idx_hbm.at[pl.ds(TAIL0, TAILW)], cidxt)
            for g in range(2):
                pidxt[pl.ds(16 * g, 16)] = cidxt[pl.ds(16 * g, 16)] >> 1
            pltpu.async_copy(pairs_hbm.at[pidxt], gtail, tsem).wait()
            h64t = tuple(
                (cidxt[pl.ds(16 * g, 16)] & 1) << 6 for g in range(2)
            )

            @plsc.parallel_loop(0, D, unroll=4)
            def _(f):
                for g in range(2):
                    otail[f, pl.ds(16 * g, 16)] = plsc.load_gather(
                        gtail, [rows8[g], h64t[g] + f]
                    )
            pltpu.sync_copy(otail, out_hbm.at[:, pl.ds(TAIL0, TAILW)])

    return gather_kernel


def kernel(x, emb_weight):
    idx = x[:, 0].astype(jnp.int32)
    wt = emb_weight.T
    pairs = _ka()(wt)
    out_t = _kb()(pairs, idx)
    return out_t.T


# speedup vs baseline: 1.7424x; 1.0003x over previous
import functools

import jax
import jax.numpy as jnp
from jax import lax
from jax.experimental import pallas as pl
from jax.experimental.pallas import tpu as pltpu
from jax.experimental.pallas import tpu_sc as plsc

V = 100000
D = 64
NB = 781
TAIL0 = NB * 128
TAILW = V - TAIL0
P = V // 2
NC = 2
NS = 16
NW = NC * NS
NBIG = NB - NW * (NB // NW)
CP = pltpu.CompilerParams(use_tc_tiling_on_sc=True, needs_layout_passes=False)


def _worker_range(wid):
    n = jnp.where(wid < NBIG, NB // NW + 1, NB // NW)
    start = jnp.where(
        wid < NBIG, (NB // NW + 1) * wid, NB // NW * wid + NBIG
    )
    return start, n


def _iota16():
    return lax.iota(jnp.int32, 16)


@functools.cache
def _ka():
    mesh = plsc.VectorSubcoreMesh(core_axis_name="c", subcore_axis_name="s")

    @functools.partial(
        pl.kernel,
        mesh=mesh,
        out_type=jax.ShapeDtypeStruct((P, 128), jnp.float32),
        scratch_types=[
            pltpu.VMEM((D, 128), jnp.float32),
            pltpu.VMEM((D, 128), jnp.float32),
            pltpu.VMEM((D, 128), jnp.float32),
            pltpu.VMEM((D, 128), jnp.float32),
            pltpu.VMEM((D, TAILW), jnp.float32),
            pltpu.VMEM((TAILW // 2, 128), jnp.float32),
            pltpu.SemaphoreType.DMA,
            pltpu.SemaphoreType.DMA,
            pltpu.SemaphoreType.DMA,
            pltpu.SemaphoreType.DMA,
        ],
        compiler_params=CP,
    )
    def transpose_kernel(
        wt_hbm, pairs_hbm, a0, a1, b0, b1, atail, btail, g0, g1, w0, w1
    ):
        wid = lax.axis_index("s") * NC + lax.axis_index("c")
        start, n = _worker_range(wid)
        ablk = (a0, a1)
        bblk = (b0, b1)
        gsem = (g0, g1)
        wsem = (w0, w1)
        rows4 = [_iota16() + 16 * t for t in range(4)]

        def in_dma(k, par):
            return pltpu.async_copy(
                wt_hbm.at[:, pl.ds((start + k) * 128, 128)], ablk[par], gsem[par]
            )

        def out_dma(k, par):
            return pltpu.async_copy(
                bblk[par], pairs_hbm.at[pl.ds((start + k) * 64, 64)], wsem[par]
            )

        def permute(src, dst, q):
            ce = jnp.full((16,), 2 * q, jnp.int32)
            co = ce + 1
            for t in range(4):
                dst[q, pl.ds(16 * t, 16)] = plsc.load_gather(src, [rows4[t], ce])
                dst[q, pl.ds(64 + 16 * t, 16)] = plsc.load_gather(
                    src, [rows4[t], co]
                )

        in_dma(0, 0)

        @pl.when(n > 1)
        def _():
            in_dma(1, 1)

        def body(i, carry):
            for par in range(2):
                k = 2 * i + par

                @pl.when(k < n)
                def _():
                    pltpu.make_async_copy(
                        wt_hbm.at[:, pl.ds(0, 128)], ablk[par], gsem[par]
                    ).wait()

                    @pl.when(k >= 2)
                    def _():
                        pltpu.make_async_copy(
                            bblk[par], pairs_hbm.at[pl.ds(0, 64)], wsem[par]
                        ).wait()

                    @plsc.parallel_loop(0, D, unroll=8)
                    def _(q):
                        permute(ablk[par], bblk[par], q)

                    out_dma(k, par)

                    @pl.when(k + 2 < n)
                    def _():
                        in_dma(k + 2, par)

            return carry

        lax.fori_loop(0, 13, body, 0)
        for par in range(2):

            @pl.when(n > par)
            def _():
                pltpu.make_async_copy(
                    bblk[par], pairs_hbm.at[pl.ds(0, 64)], wsem[par]
                ).wait()

        @pl.when(wid == NW - 1)
        def _():
            pltpu.sync_copy(wt_hbm.at[:, pl.ds(TAIL0, TAILW)], atail)

            @plsc.parallel_loop(0, TAILW // 2, unroll=8)
            def _(q):
                ce = jnp.full((16,), 2 * q, jnp.int32)
                co = ce + 1
                for t in range(4):
                    btail[q, pl.ds(16 * t, 16)] = plsc.load_gather(
                        atail, [rows4[t], ce]
                    )
                    btail[q, pl.ds(64 + 16 * t, 16)] = plsc.load_gather(
                        atail, [rows4[t], co]
                    )
            pltpu.sync_copy(btail, pairs_hbm.at[pl.ds(P - TAILW // 2, TAILW // 2)])

    return transpose_kernel


@functools.cache
def _kb():
    mesh = plsc.VectorSubcoreMesh(core_axis_name="c", subcore_axis_name="s")

    @functools.partial(
        pl.kernel,
        mesh=mesh,
        out_type=jax.ShapeDtypeStruct((D, V), jnp.float32),
        scratch_types=[
            pltpu.VMEM((3200,), jnp.int32),
            pltpu.VMEM((128,), jnp.int32),
            pltpu.VMEM((128,), jnp.int32),
            pltpu.VMEM((128, 128), jnp.float32),
            pltpu.VMEM((128, 128), jnp.float32),
            pltpu.VMEM((D, 128), jnp.float32),
            pltpu.VMEM((D, 128), jnp.float32),
            pltpu.VMEM((TAILW,), jnp.int32),
            pltpu.VMEM((TAILW,), jnp.int32),
            pltpu.VMEM((TAILW, 128), jnp.float32),
            pltpu.VMEM((D, TAILW), jnp.float32),
            pltpu.SemaphoreType.DMA,
            pltpu.SemaphoreType.DMA,
            pltpu.SemaphoreType.DMA,
            pltpu.SemaphoreType.DMA,
            pltpu.SemaphoreType.DMA,
        ],
        compiler_params=CP,
    )
    def gather_kernel(
        pairs_hbm,
        idx_hbm,
        out_hbm,
        cidx,
        p0,
        p1,
        gb0,
        gb1,
        ob0,
        ob1,
        cidxt,
        pidxt,
        gtail,
        otail,
        g0,
        g1,
        w0,
        w1,
        tsem,
    ):
        wid = lax.axis_index("s") * NC + lax.axis_index("c")
        start, n = _worker_range(wid)
        bufst = jnp.minimum(start, NB - 25)
        lo = start - bufst
        pidx = (p0, p1)
        gblk = (gb0, gb1)
        oblk = (ob0, ob1)
        gsem = (g0, g1)
        wsem = (w0, w1)
        rows8 = [_iota16() + 16 * g for g in range(8)]

        pltpu.sync_copy(idx_hbm.at[pl.ds(bufst * 128, 3200)], cidx)

        def compute_pidx_and_fire(k, par):
            base = (lo + k) * 128
            for g in range(8):
                pidx[par][pl.ds(16 * g, 16)] = (
                    cidx[pl.ds(base + 16 * g, 16)] >> 1
                )
            pltpu.async_copy(pairs_hbm.at[pidx[par]], gblk[par], gsem[par])

        compute_pidx_and_fire(0, 0)

        @pl.when(n > 1)
        def _():
            compute_pidx_and_fire(1, 1)

        def body(i, carry):
            for par in range(2):
                k = 2 * i + par

                @pl.when(k < n)
                def _():
                    pltpu.make_async_copy(
                        pairs_hbm.at[pidx[par]], gblk[par], gsem[par]
                    ).wait()

                    @pl.when(k >= 2)
                    def _():
                        pltpu.make_async_copy(
                            oblk[par], out_hbm.at[:, pl.ds(0, 128)], wsem[par]
                        ).wait()

                    base = (lo + k) * 128
                    h64 = tuple(
                        (cidx[pl.ds(base + 16 * g, 16)] & 1) << 6
                        for g in range(8)
                    )

                    @plsc.parallel_loop(0, D, unroll=8)
                    def _(f):
                        for g in range(8):
                            ob = oblk[par]
                            ob[f, pl.ds(16 * g, 16)] = plsc.load_gather(
                                gblk[par], [rows8[g], h64[g] + f]
                            )
                    pltpu.async_copy(
                        oblk[par],
                        out_hbm.at[:, pl.ds((start + k) * 128, 128)],
                        wsem[par],
                    )

                    @pl.when(k + 2 < n)
                    def _():
                        compute_pidx_and_fire(k + 2, par)

            return carry

        lax.fori_loop(0, 13, body, 0)
        for par in range(2):

            @pl.when(n > par)
            def _():
                pltpu.make_async_copy(
                    oblk[par], out_hbm.at[:, pl.ds(0, 128)], wsem[par]
                ).wait()

        @pl.when(wid == NW - 1)
        def _():
            pltpu.sync_copy(idx_hbm.at[pl.ds(TAIL0, TAILW)], cidxt)
            for g in range(2):
                pidxt[pl.ds(16 * g, 16)] = cidxt[pl.ds(16 * g, 16)] >> 1
            pltpu.async_copy(pairs_hbm.at[pidxt], gtail, tsem).wait()
            h64t = tuple(
                (cidxt[pl.ds(16 * g, 16)] & 1) << 6 for g in range(2)
            )

            @plsc.parallel_loop(0, D, unroll=8)
            def _(f):
                for g in range(2):
                    otail[f, pl.ds(16 * g, 16)] = plsc.load_gather(
                        gtail, [rows8[g], h64t[g] + f]
                    )
            pltpu.sync_copy(otail, out_hbm.at[:, pl.ds(TAIL0, TAILW)])

    return gather_kernel


def kernel(x, emb_weight):
    idx = x[:, 0].astype(jnp.int32)
    wt = emb_weight.T
    pairs = _ka()(wt)
    out_t = _kb()(pairs, idx)
    return out_t.T


# speedup vs baseline: 3.8914x; 2.2334x over previous
import functools

import jax
import jax.numpy as jnp
from jax import lax
from jax.experimental import pallas as pl
from jax.experimental.pallas import tpu as pltpu
from jax.experimental.pallas import tpu_sc as plsc

V = 100000
D = 64
NB = 781
TAIL0 = NB * 128
TAILW = V - TAIL0
P = V // 2
NC = 2
NS = 16
NW = NC * NS
NBIG = NB - NW * (NB // NW)
CP = pltpu.CompilerParams(use_tc_tiling_on_sc=True, needs_layout_passes=False)


def _worker_range(wid):
    n = jnp.where(wid < NBIG, NB // NW + 1, NB // NW)
    start = jnp.where(
        wid < NBIG, (NB // NW + 1) * wid, NB // NW * wid + NBIG
    )
    return start, n


def _iota16():
    return lax.iota(jnp.int32, 16)


def _diag_vecs():
    io = _iota16()
    return io, io ^ 1, io // 2, io % 2


@functools.cache
def _ka():
    mesh = plsc.VectorSubcoreMesh(core_axis_name="c", subcore_axis_name="s")

    @functools.partial(
        pl.kernel,
        mesh=mesh,
        out_type=jax.ShapeDtypeStruct((P, 128), jnp.float32),
        scratch_types=[
            pltpu.VMEM((D, 128), jnp.float32),
            pltpu.VMEM((D, 128), jnp.float32),
            pltpu.VMEM((D, 128), jnp.float32),
            pltpu.VMEM((D, 128), jnp.float32),
            pltpu.VMEM((D, TAILW), jnp.float32),
            pltpu.VMEM((TAILW // 2, 128), jnp.float32),
            pltpu.SemaphoreType.DMA,
            pltpu.SemaphoreType.DMA,
            pltpu.SemaphoreType.DMA,
            pltpu.SemaphoreType.DMA,
        ],
        compiler_params=CP,
    )
    def transpose_kernel(
        wt_hbm, pairs_hbm, a0, a1, b0, b1, atail, btail, g0, g1, w0, w1
    ):
        wid = lax.axis_index("s") * NC + lax.axis_index("c")
        start, n = _worker_range(wid)
        ablk = (a0, a1)
        bblk = (b0, b1)
        gsem = (g0, g1)
        wsem = (w0, w1)
        io, ix, i2, m2 = _diag_vecs()

        def in_dma(k, par):
            return pltpu.async_copy(
                wt_hbm.at[:, pl.ds((start + k) * 128, 128)], ablk[par], gsem[par]
            )

        def out_dma(k, par):
            return pltpu.async_copy(
                bblk[par], pairs_hbm.at[pl.ds((start + k) * 64, 64)], wsem[par]
            )

        def permute(src, dst, q0, qmask):
            qv = (q0 + i2) & qmask
            cv = 2 * qv + m2
            for t in range(4):
                for rv in (16 * t + io, 16 * t + ix):
                    wv = rv + 64 * m2
                    vals = plsc.load_gather(src, [rv, cv])
                    plsc.store_scatter(dst, [qv, wv], vals)

        in_dma(0, 0)

        @pl.when(n > 1)
        def _():
            in_dma(1, 1)

        def body(i, carry):
            for par in range(2):
                k = 2 * i + par

                @pl.when(k < n)
                def _():
                    pltpu.make_async_copy(
                        wt_hbm.at[:, pl.ds(0, 128)], ablk[par], gsem[par]
                    ).wait()

                    @pl.when(k >= 2)
                    def _():
                        pltpu.make_async_copy(
                            bblk[par], pairs_hbm.at[pl.ds(0, 64)], wsem[par]
                        ).wait()

                    @plsc.parallel_loop(0, D, unroll=8)
                    def _(q0):
                        permute(ablk[par], bblk[par], q0, 63)

                    out_dma(k, par)

                    @pl.when(k + 2 < n)
                    def _():
                        in_dma(k + 2, par)

            return carry

        lax.fori_loop(0, 13, body, 0)
        for par in range(2):

            @pl.when(n > par)
            def _():
                pltpu.make_async_copy(
                    bblk[par], pairs_hbm.at[pl.ds(0, 64)], wsem[par]
                ).wait()

        @pl.when(wid == NW - 1)
        def _():
            pltpu.sync_copy(wt_hbm.at[:, pl.ds(TAIL0, TAILW)], atail)

            @plsc.parallel_loop(0, TAILW // 2, unroll=8)
            def _(q0):
                permute(atail, btail, q0, TAILW // 2 - 1)

            pltpu.sync_copy(btail, pairs_hbm.at[pl.ds(P - TAILW // 2, TAILW // 2)])

    return transpose_kernel


@functools.cache
def _kb():
    mesh = plsc.VectorSubcoreMesh(core_axis_name="c", subcore_axis_name="s")

    @functools.partial(
        pl.kernel,
        mesh=mesh,
        out_type=jax.ShapeDtypeStruct((D, V), jnp.float32),
        scratch_types=[
            pltpu.VMEM((3200,), jnp.int32),
            pltpu.VMEM((128,), jnp.int32),
            pltpu.VMEM((128,), jnp.int32),
            pltpu.VMEM((128, 128), jnp.float32),
            pltpu.VMEM((128, 128), jnp.float32),
            pltpu.VMEM((D, 128), jnp.float32),
            pltpu.VMEM((D, 128), jnp.float32),
            pltpu.VMEM((TAILW,), jnp.int32),
            pltpu.VMEM((TAILW,), jnp.int32),
            pltpu.VMEM((TAILW, 128), jnp.float32),
            pltpu.VMEM((D, TAILW), jnp.float32),
            pltpu.SemaphoreType.DMA,
            pltpu.SemaphoreType.DMA,
            pltpu.SemaphoreType.DMA,
            pltpu.SemaphoreType.DMA,
            pltpu.SemaphoreType.DMA,
        ],
        compiler_params=CP,
    )
    def gather_kernel(
        pairs_hbm,
        idx_hbm,
        out_hbm,
        cidx,
        p0,
        p1,
        gb0,
        gb1,
        ob0,
        ob1,
        cidxt,
        pidxt,
        gtail,
        otail,
        g0,
        g1,
        w0,
        w1,
        tsem,
    ):
        wid = lax.axis_index("s") * NC + lax.axis_index("c")
        start, n = _worker_range(wid)
        bufst = jnp.minimum(start, NB - 25)
        lo = start - bufst
        pidx = (p0, p1)
        gblk = (gb0, gb1)
        oblk = (ob0, ob1)
        gsem = (g0, g1)
        wsem = (w0, w1)
        io = _iota16()

        pltpu.sync_copy(idx_hbm.at[pl.ds(bufst * 128, 3200)], cidx)

        def compute_pidx_and_fire(k, par):
            base = (lo + k) * 128
            for g in range(8):
                pidx[par][pl.ds(16 * g, 16)] = (
                    cidx[pl.ds(base + 16 * g, 16)] >> 1
                )
            pltpu.async_copy(pairs_hbm.at[pidx[par]], gblk[par], gsem[par])

        compute_pidx_and_fire(0, 0)

        @pl.when(n > 1)
        def _():
            compute_pidx_and_fire(1, 1)

        def body(i, carry):
            for par in range(2):
                k = 2 * i + par

                @pl.when(k < n)
                def _():
                    pltpu.make_async_copy(
                        pairs_hbm.at[pidx[par]], gblk[par], gsem[par]
                    ).wait()

                    @pl.when(k >= 2)
                    def _():
                        pltpu.make_async_copy(
                            oblk[par], out_hbm.at[:, pl.ds(0, 128)], wsem[par]
                        ).wait()

                    base = (lo + k) * 128
                    h64 = tuple(
                        (cidx[pl.ds(base + 16 * g, 16)] & 1) << 6
                        for g in range(8)
                    )

                    @plsc.parallel_loop(0, D, unroll=8)
                    def _(f0):
                        fv = (f0 + io) & 63
                        for g in range(8):
                            vals = plsc.load_gather(
                                gblk[par], [16 * g + io, h64[g] + fv]
                            )
                            plsc.store_scatter(
                                oblk[par], [fv, 16 * g + io], vals
                            )
                    pltpu.async_copy(
                        oblk[par],
                        out_hbm.at[:, pl.ds((start + k) * 128, 128)],
                        wsem[par],
                    )

                    @pl.when(k + 2 < n)
                    def _():
                        compute_pidx_and_fire(k + 2, par)

            return carry

        lax.fori_loop(0, 13, body, 0)
        for par in range(2):

            @pl.when(n > par)
            def _():
                pltpu.make_async_copy(
                    oblk[par], out_hbm.at[:, pl.ds(0, 128)], wsem[par]
                ).wait()

        @pl.when(wid == NW - 1)
        def _():
            pltpu.sync_copy(idx_hbm.at[pl.ds(TAIL0, TAILW)], cidxt)
            for g in range(2):
                pidxt[pl.ds(16 * g, 16)] = cidxt[pl.ds(16 * g, 16)] >> 1
            pltpu.async_copy(pairs_hbm.at[pidxt], gtail, tsem).wait()
            h64t = tuple(
                (cidxt[pl.ds(16 * g, 16)] & 1) << 6 for g in range(2)
            )

            @plsc.parallel_loop(0, D, unroll=8)
            def _(f0):
                fv = (f0 + io) & 63
                for g in range(2):
                    vals = plsc.load_gather(
                        gtail, [16 * g + io, h64t[g] + fv]
                    )
                    plsc.store_scatter(otail, [fv, 16 * g + io], vals)
            pltpu.sync_copy(otail, out_hbm.at[:, pl.ds(TAIL0, TAILW)])

    return gather_kernel


def kernel(x, emb_weight):
    idx = x[:, 0].astype(jnp.int32)
    wt = emb_weight.T
    pairs = _ka()(wt)
    out_t = _kb()(pairs, idx)
    return out_t.T


# speedup vs baseline: 3.9258x; 1.0088x over previous
import functools

import jax
import jax.numpy as jnp
from jax import lax
from jax.experimental import pallas as pl
from jax.experimental.pallas import tpu as pltpu
from jax.experimental.pallas import tpu_sc as plsc

V = 100000
D = 64
NB = 781
TAIL0 = NB * 128
TAILW = V - TAIL0
P = V // 2
NC = 2
NS = 16
NW = NC * NS
NBIG = NB - NW * (NB // NW)
CP = pltpu.CompilerParams(use_tc_tiling_on_sc=True, needs_layout_passes=False)


def _worker_range(wid):
    n = jnp.where(wid < NBIG, NB // NW + 1, NB // NW)
    start = jnp.where(
        wid < NBIG, (NB // NW + 1) * wid, NB // NW * wid + NBIG
    )
    return start, n


def _iota16():
    return lax.iota(jnp.int32, 16)


def _diag_vecs():
    io = _iota16()
    return io, io ^ 1, io // 2, io % 2


@functools.cache
def _ka():
    mesh = plsc.VectorSubcoreMesh(core_axis_name="c", subcore_axis_name="s")

    @functools.partial(
        pl.kernel,
        mesh=mesh,
        out_type=jax.ShapeDtypeStruct((P, 128), jnp.float32),
        scratch_types=[
            pltpu.VMEM((D, 128), jnp.float32),
            pltpu.VMEM((D, 128), jnp.float32),
            pltpu.VMEM((D, 128), jnp.float32),
            pltpu.VMEM((D, 128), jnp.float32),
            pltpu.VMEM((D, TAILW), jnp.float32),
            pltpu.VMEM((TAILW // 2, 128), jnp.float32),
            pltpu.SemaphoreType.DMA,
            pltpu.SemaphoreType.DMA,
            pltpu.SemaphoreType.DMA,
            pltpu.SemaphoreType.DMA,
        ],
        compiler_params=CP,
    )
    def transpose_kernel(
        wt_hbm, pairs_hbm, a0, a1, b0, b1, atail, btail, g0, g1, w0, w1
    ):
        wid = lax.axis_index("s") * NC + lax.axis_index("c")
        start, n = _worker_range(wid)
        ablk = (a0, a1)
        bblk = (b0, b1)
        gsem = (g0, g1)
        wsem = (w0, w1)
        io, ix, i2, m2 = _diag_vecs()

        def in_dma(k, par):
            return pltpu.async_copy(
                wt_hbm.at[:, pl.ds((start + k) * 128, 128)], ablk[par], gsem[par]
            )

        def out_dma(k, par):
            return pltpu.async_copy(
                bblk[par], pairs_hbm.at[pl.ds((start + k) * 64, 64)], wsem[par]
            )

        def permute(src, dst, q0, qmask):
            qv = (q0 + i2) & qmask
            cv = 2 * qv + m2
            for t in range(4):
                for rv in (16 * t + io, 16 * t + ix):
                    wv = rv + 64 * m2
                    vals = plsc.load_gather(src, [rv, cv])
                    plsc.store_scatter(dst, [qv, wv], vals)

        in_dma(0, 0)

        @pl.when(n > 1)
        def _():
            in_dma(1, 1)

        def body(i, carry):
            for par in range(2):
                k = 2 * i + par

                @pl.when(k < n)
                def _():
                    pltpu.make_async_copy(
                        wt_hbm.at[:, pl.ds(0, 128)], ablk[par], gsem[par]
                    ).wait()

                    @pl.when(k >= 2)
                    def _():
                        pltpu.make_async_copy(
                            bblk[par], pairs_hbm.at[pl.ds(0, 64)], wsem[par]
                        ).wait()

                    @plsc.parallel_loop(0, D, unroll=8)
                    def _(q0):
                        permute(ablk[par], bblk[par], q0, 63)

                    out_dma(k, par)

                    @pl.when(k + 2 < n)
                    def _():
                        in_dma(k + 2, par)

            return carry

        lax.fori_loop(0, 13, body, 0)
        for par in range(2):

            @pl.when(n > par)
            def _():
                pltpu.make_async_copy(
                    bblk[par], pairs_hbm.at[pl.ds(0, 64)], wsem[par]
                ).wait()

        @pl.when(wid == NW - 1)
        def _():
            pltpu.sync_copy(wt_hbm.at[:, pl.ds(TAIL0, TAILW)], atail)

            @plsc.parallel_loop(0, TAILW // 2, unroll=8)
            def _(q0):
                permute(atail, btail, q0, TAILW // 2 - 1)

            pltpu.sync_copy(btail, pairs_hbm.at[pl.ds(P - TAILW // 2, TAILW // 2)])

    return transpose_kernel


@functools.cache
def _kb():
    mesh = plsc.VectorSubcoreMesh(core_axis_name="c", subcore_axis_name="s")

    @functools.partial(
        pl.kernel,
        mesh=mesh,
        out_type=jax.ShapeDtypeStruct((D, V), jnp.float32),
        scratch_types=[
            pltpu.VMEM((3200,), jnp.int32),
            pltpu.VMEM((128,), jnp.int32),
            pltpu.VMEM((128,), jnp.int32),
            pltpu.VMEM((128,), jnp.int32),
            pltpu.VMEM((128, 128), jnp.float32),
            pltpu.VMEM((128, 128), jnp.float32),
            pltpu.VMEM((128, 128), jnp.float32),
            pltpu.VMEM((D, 128), jnp.float32),
            pltpu.VMEM((D, 128), jnp.float32),
            pltpu.VMEM((TAILW,), jnp.int32),
            pltpu.VMEM((TAILW,), jnp.int32),
            pltpu.VMEM((TAILW, 128), jnp.float32),
            pltpu.VMEM((D, TAILW), jnp.float32),
            pltpu.SemaphoreType.DMA,
            pltpu.SemaphoreType.DMA,
            pltpu.SemaphoreType.DMA,
            pltpu.SemaphoreType.DMA,
            pltpu.SemaphoreType.DMA,
            pltpu.SemaphoreType.DMA,
        ],
        compiler_params=CP,
    )
    def gather_kernel(
        pairs_hbm,
        idx_hbm,
        out_hbm,
        cidx,
        p0,
        p1,
        p2,
        gb0,
        gb1,
        gb2,
        ob0,
        ob1,
        cidxt,
        pidxt,
        gtail,
        otail,
        g0,
        g1,
        g2,
        w0,
        w1,
        tsem,
    ):
        wid = lax.axis_index("s") * NC + lax.axis_index("c")
        start, n = _worker_range(wid)
        bufst = jnp.minimum(start, NB - 25)
        lo = start - bufst
        pidx = (p0, p1, p2)
        gblk = (gb0, gb1, gb2)
        oblk = (ob0, ob1)
        gsem = (g0, g1, g2)
        wsem = (w0, w1)
        io = _iota16()

        pltpu.sync_copy(idx_hbm.at[pl.ds(bufst * 128, 3200)], cidx)

        def compute_pidx_and_fire(k, par):
            base = (lo + k) * 128
            for g in range(8):
                pidx[par][pl.ds(16 * g, 16)] = (
                    cidx[pl.ds(base + 16 * g, 16)] >> 1
                )
            pltpu.async_copy(pairs_hbm.at[pidx[par]], gblk[par], gsem[par])

        compute_pidx_and_fire(0, 0)

        @pl.when(n > 1)
        def _():
            compute_pidx_and_fire(1, 1)

        @pl.when(n > 2)
        def _():
            compute_pidx_and_fire(2, 2)

        def body(i, carry):
            for j in range(6):
                k = 6 * i + j
                p3 = j % 3
                pr = j % 2

                @pl.when(k < n)
                def _():
                    pltpu.make_async_copy(
                        pairs_hbm.at[pidx[p3]], gblk[p3], gsem[p3]
                    ).wait()

                    @pl.when(k >= 2)
                    def _():
                        pltpu.make_async_copy(
                            oblk[pr], out_hbm.at[:, pl.ds(0, 128)], wsem[pr]
                        ).wait()

                    base = (lo + k) * 128
                    h64 = tuple(
                        (cidx[pl.ds(base + 16 * g, 16)] & 1) << 6
                        for g in range(8)
                    )

                    @plsc.parallel_loop(0, D, unroll=8)
                    def _(f0):
                        fv = (f0 + io) & 63
                        for g in range(8):
                            vals = plsc.load_gather(
                                gblk[p3], [16 * g + io, h64[g] + fv]
                            )
                            plsc.store_scatter(
                                oblk[pr], [fv, 16 * g + io], vals
                            )
                    pltpu.async_copy(
                        oblk[pr],
                        out_hbm.at[:, pl.ds((start + k) * 128, 128)],
                        wsem[pr],
                    )

                    @pl.when(k + 3 < n)
                    def _():
                        compute_pidx_and_fire(k + 3, p3)

            return carry

        lax.fori_loop(0, 5, body, 0)
        for par in range(2):

            @pl.when(n > par)
            def _():
                pltpu.make_async_copy(
                    oblk[par], out_hbm.at[:, pl.ds(0, 128)], wsem[par]
                ).wait()

        @pl.when(wid == NW - 1)
        def _():
            pltpu.sync_copy(idx_hbm.at[pl.ds(TAIL0, TAILW)], cidxt)
            for g in range(2):
                pidxt[pl.ds(16 * g, 16)] = cidxt[pl.ds(16 * g, 16)] >> 1
            pltpu.async_copy(pairs_hbm.at[pidxt], gtail, tsem).wait()
            h64t = tuple(
                (cidxt[pl.ds(16 * g, 16)] & 1) << 6 for g in range(2)
            )

            @plsc.parallel_loop(0, D, unroll=8)
            def _(f0):
                fv = (f0 + io) & 63
                for g in range(2):
                    vals = plsc.load_gather(
                        gtail, [16 * g + io, h64t[g] + fv]
                    )
                    plsc.store_scatter(otail, [fv, 16 * g + io], vals)
            pltpu.sync_copy(otail, out_hbm.at[:, pl.ds(TAIL0, TAILW)])

    return gather_kernel


def kernel(x, emb_weight):
    idx = x[:, 0].astype(jnp.int32)
    wt = emb_weight.T
    pairs = _ka()(wt)
    out_t = _kb()(pairs, idx)
    return out_t.T


# speedup vs baseline: 4.2315x; 1.0779x over previous
import functools

import jax
import jax.numpy as jnp
from jax import lax
from jax.experimental import pallas as pl
from jax.experimental.pallas import tpu as pltpu
from jax.experimental.pallas import tpu_sc as plsc

V = 100000
D = 64
NB = 781
TAIL0 = NB * 128
TAILW = V - TAIL0
P = V // 2
NC = 2
NS = 16
NW = NC * NS
NBIG = NB - NW * (NB // NW)
CP = pltpu.CompilerParams(use_tc_tiling_on_sc=True, needs_layout_passes=False)


def _worker_range(wid):
    n = jnp.where(wid < NBIG, NB // NW + 1, NB // NW)
    start = jnp.where(
        wid < NBIG, (NB // NW + 1) * wid, NB // NW * wid + NBIG
    )
    return start, n


def _iota16():
    return lax.iota(jnp.int32, 16)


def _diag_vecs():
    io = _iota16()
    return io, io ^ 1, io // 2, io % 2


@functools.cache
def _ka():
    mesh = plsc.VectorSubcoreMesh(core_axis_name="c", subcore_axis_name="s")

    @functools.partial(
        pl.kernel,
        mesh=mesh,
        out_type=jax.ShapeDtypeStruct((P, 128), jnp.float32),
        scratch_types=[
            pltpu.VMEM((D, 128), jnp.float32),
            pltpu.VMEM((D, 128), jnp.float32),
            pltpu.VMEM((D, 128), jnp.float32),
            pltpu.VMEM((D, 128), jnp.float32),
            pltpu.VMEM((D, TAILW), jnp.float32),
            pltpu.VMEM((TAILW // 2, 128), jnp.float32),
            pltpu.SemaphoreType.DMA,
            pltpu.SemaphoreType.DMA,
            pltpu.SemaphoreType.DMA,
            pltpu.SemaphoreType.DMA,
        ],
        compiler_params=CP,
    )
    def transpose_kernel(
        wt_hbm, pairs_hbm, a0, a1, b0, b1, atail, btail, g0, g1, w0, w1
    ):
        wid = lax.axis_index("s") * NC + lax.axis_index("c")
        start, n = _worker_range(wid)
        ablk = (a0, a1)
        bblk = (b0, b1)
        gsem = (g0, g1)
        wsem = (w0, w1)
        io, ix, i2, m2 = _diag_vecs()

        def in_dma(k, par):
            return pltpu.async_copy(
                wt_hbm.at[:, pl.ds((start + k) * 128, 128)], ablk[par], gsem[par]
            )

        def out_dma(k, par):
            return pltpu.async_copy(
                bblk[par], pairs_hbm.at[pl.ds((start + k) * 64, 64)], wsem[par]
            )

        def permute(src, dst, q0, qmask):
            qv = (q0 + i2) & qmask
            cv = 2 * qv + m2
            for t in range(4):
                for rv in (16 * t + io, 16 * t + ix):
                    wv = rv + 64 * m2
                    vals = plsc.load_gather(src, [rv, cv])
                    plsc.store_scatter(dst, [qv, wv], vals)

        in_dma(0, 0)

        @pl.when(n > 1)
        def _():
            in_dma(1, 1)

        def body(i, carry):
            for par in range(2):
                k = 2 * i + par

                @pl.when(k < n)
                def _():
                    pltpu.make_async_copy(
                        wt_hbm.at[:, pl.ds(0, 128)], ablk[par], gsem[par]
                    ).wait()

                    @pl.when(k >= 2)
                    def _():
                        pltpu.make_async_copy(
                            bblk[par], pairs_hbm.at[pl.ds(0, 64)], wsem[par]
                        ).wait()

                    @plsc.parallel_loop(0, D, unroll=4)
                    def _(q0):
                        permute(ablk[par], bblk[par], q0, 63)

                    out_dma(k, par)

                    @pl.when(k + 2 < n)
                    def _():
                        in_dma(k + 2, par)

            return carry

        lax.fori_loop(0, 13, body, 0)
        for par in range(2):

            @pl.when(n > par)
            def _():
                pltpu.make_async_copy(
                    bblk[par], pairs_hbm.at[pl.ds(0, 64)], wsem[par]
                ).wait()

        @pl.when(wid == NW - 1)
        def _():
            pltpu.sync_copy(wt_hbm.at[:, pl.ds(TAIL0, TAILW)], atail)

            @plsc.parallel_loop(0, TAILW // 2, unroll=4)
            def _(q0):
                permute(atail, btail, q0, TAILW // 2 - 1)

            pltpu.sync_copy(btail, pairs_hbm.at[pl.ds(P - TAILW // 2, TAILW // 2)])

    return transpose_kernel


@functools.cache
def _kb():
    mesh = plsc.VectorSubcoreMesh(core_axis_name="c", subcore_axis_name="s")

    @functools.partial(
        pl.kernel,
        mesh=mesh,
        out_type=jax.ShapeDtypeStruct((D, V), jnp.float32),
        scratch_types=[
            pltpu.VMEM((3200,), jnp.int32),
            pltpu.VMEM((128,), jnp.int32),
            pltpu.VMEM((128,), jnp.int32),
            pltpu.VMEM((128,), jnp.int32),
            pltpu.VMEM((128, 128), jnp.float32),
            pltpu.VMEM((128, 128), jnp.float32),
            pltpu.VMEM((128, 128), jnp.float32),
            pltpu.VMEM((D, 128), jnp.float32),
            pltpu.VMEM((D, 128), jnp.float32),
            pltpu.VMEM((TAILW,), jnp.int32),
            pltpu.VMEM((TAILW,), jnp.int32),
            pltpu.VMEM((TAILW, 128), jnp.float32),
            pltpu.VMEM((D, TAILW), jnp.float32),
            pltpu.SemaphoreType.DMA,
            pltpu.SemaphoreType.DMA,
            pltpu.SemaphoreType.DMA,
            pltpu.SemaphoreType.DMA,
            pltpu.SemaphoreType.DMA,
            pltpu.SemaphoreType.DMA,
        ],
        compiler_params=CP,
    )
    def gather_kernel(
        pairs_hbm,
        idx_hbm,
        out_hbm,
        cidx,
        p0,
        p1,
        p2,
        gb0,
        gb1,
        gb2,
        ob0,
        ob1,
        cidxt,
        pidxt,
        gtail,
        otail,
        g0,
        g1,
        g2,
        w0,
        w1,
        tsem,
    ):
        wid = lax.axis_index("s") * NC + lax.axis_index("c")
        start, n = _worker_range(wid)
        bufst = jnp.minimum(start, NB - 25)
        lo = start - bufst
        pidx = (p0, p1, p2)
        gblk = (gb0, gb1, gb2)
        oblk = (ob0, ob1)
        gsem = (g0, g1, g2)
        wsem = (w0, w1)
        io = _iota16()

        pltpu.sync_copy(idx_hbm.at[pl.ds(bufst * 128, 3200)], cidx)

        def compute_pidx_and_fire(k, par):
            base = (lo + k) * 128
            for g in range(8):
                pidx[par][pl.ds(16 * g, 16)] = (
                    cidx[pl.ds(base + 16 * g, 16)] >> 1
                )
            pltpu.async_copy(pairs_hbm.at[pidx[par]], gblk[par], gsem[par])

        compute_pidx_and_fire(0, 0)

        @pl.when(n > 1)
        def _():
            compute_pidx_and_fire(1, 1)

        @pl.when(n > 2)
        def _():
            compute_pidx_and_fire(2, 2)

        def body(i, carry):
            for j in range(6):
                k = 6 * i + j
                p3 = j % 3
                pr = j % 2

                @pl.when(k < n)
                def _():
                    pltpu.make_async_copy(
                        pairs_hbm.at[pidx[p3]], gblk[p3], gsem[p3]
                    ).wait()

                    @pl.when(k >= 2)
                    def _():
                        pltpu.make_async_copy(
                            oblk[pr], out_hbm.at[:, pl.ds(0, 128)], wsem[pr]
                        ).wait()

                    base = (lo + k) * 128
                    h64 = tuple(
                        (cidx[pl.ds(base + 16 * g, 16)] & 1) << 6
                        for g in range(8)
                    )

                    @plsc.parallel_loop(0, D, unroll=4)
                    def _(f0):
                        fv = (f0 + io) & 63
                        for g in range(8):
                            vals = plsc.load_gather(
                                gblk[p3], [16 * g + io, h64[g] + fv]
                            )
                            plsc.store_scatter(
                                oblk[pr], [fv, 16 * g + io], vals
                            )
                    pltpu.async_copy(
                        oblk[pr],
                        out_hbm.at[:, pl.ds((start + k) * 128, 128)],
                        wsem[pr],
                    )

                    @pl.when(k + 3 < n)
                    def _():
                        compute_pidx_and_fire(k + 3, p3)

            return carry

        lax.fori_loop(0, 5, body, 0)
        for par in range(2):

            @pl.when(n > par)
            def _():
                pltpu.make_async_copy(
                    oblk[par], out_hbm.at[:, pl.ds(0, 128)], wsem[par]
                ).wait()

        @pl.when(wid == NW - 1)
        def _():
            pltpu.sync_copy(idx_hbm.at[pl.ds(TAIL0, TAILW)], cidxt)
            for g in range(2):
                pidxt[pl.ds(16 * g, 16)] = cidxt[pl.ds(16 * g, 16)] >> 1
            pltpu.async_copy(pairs_hbm.at[pidxt], gtail, tsem).wait()
            h64t = tuple(
                (cidxt[pl.ds(16 * g, 16)] & 1) << 6 for g in range(2)
            )

            @plsc.parallel_loop(0, D, unroll=4)
            def _(f0):
                fv = (f0 + io) & 63
                for g in range(2):
                    vals = plsc.load_gather(
                        gtail, [16 * g + io, h64t[g] + fv]
                    )
                    plsc.store_scatter(otail, [fv, 16 * g + io], vals)
            pltpu.sync_copy(otail, out_hbm.at[:, pl.ds(TAIL0, TAILW)])

    return gather_kernel


def kernel(x, emb_weight):
    idx = x[:, 0].astype(jnp.int32)
    wt = emb_weight.T
    pairs = _ka()(wt)
    out_t = _kb()(pairs, idx)
    return out_t.T


# speedup vs baseline: 4.2836x; 1.0123x over previous
import functools

import jax
import jax.numpy as jnp
from jax import lax
from jax.experimental import pallas as pl
from jax.experimental.pallas import tpu as pltpu
from jax.experimental.pallas import tpu_sc as plsc

V = 100000
D = 64
NB = 781
TAIL0 = NB * 128
TAILW = V - TAIL0
P = V // 2
NC = 2
NS = 16
NW = NC * NS
NBIG = NB - NW * (NB // NW)
CP = pltpu.CompilerParams(use_tc_tiling_on_sc=True, needs_layout_passes=False)


def _worker_range(wid):
    n = jnp.where(wid < NBIG, NB // NW + 1, NB // NW)
    start = jnp.where(
        wid < NBIG, (NB // NW + 1) * wid, NB // NW * wid + NBIG
    )
    return start, n


def _iota16():
    return lax.iota(jnp.int32, 16)


def _diag_vecs():
    io = _iota16()
    return io, io ^ 1, io // 2, io % 2


@functools.cache
def _ka():
    mesh = plsc.VectorSubcoreMesh(core_axis_name="c", subcore_axis_name="s")

    @functools.partial(
        pl.kernel,
        mesh=mesh,
        out_type=jax.ShapeDtypeStruct((P, 128), jnp.float32),
        scratch_types=[
            pltpu.VMEM((D, 128), jnp.float32),
            pltpu.VMEM((D, 128), jnp.float32),
            pltpu.VMEM((D, 128), jnp.float32),
            pltpu.VMEM((D, 128), jnp.float32),
            pltpu.VMEM((D, TAILW), jnp.float32),
            pltpu.VMEM((TAILW // 2, 128), jnp.float32),
            pltpu.SemaphoreType.DMA,
            pltpu.SemaphoreType.DMA,
            pltpu.SemaphoreType.DMA,
            pltpu.SemaphoreType.DMA,
        ],
        compiler_params=CP,
    )
    def transpose_kernel(
        wt_hbm, pairs_hbm, a0, a1, b0, b1, atail, btail, g0, g1, w0, w1
    ):
        wid = lax.axis_index("s") * NC + lax.axis_index("c")
        start, n = _worker_range(wid)
        ablk = (a0, a1)
        bblk = (b0, b1)
        gsem = (g0, g1)
        wsem = (w0, w1)
        io, ix, i2, m2 = _diag_vecs()

        def in_dma(k, par):
            return pltpu.async_copy(
                wt_hbm.at[:, pl.ds((start + k) * 128, 128)], ablk[par], gsem[par]
            )

        def out_dma(k, par):
            return pltpu.async_copy(
                bblk[par], pairs_hbm.at[pl.ds((start + k) * 64, 64)], wsem[par]
            )

        def permute(src, dst, q0, qmask):
            qv = (q0 + i2) & qmask
            cv = 2 * qv + m2
            for t in range(4):
                for rv in (16 * t + io, 16 * t + ix):
                    wv = rv + 64 * m2
                    vals = plsc.load_gather(src, [rv, cv])
                    plsc.store_scatter(dst, [qv, wv], vals)

        in_dma(0, 0)

        @pl.when(n > 1)
        def _():
            in_dma(1, 1)

        def body(i, carry):
            for par in range(2):
                k = 2 * i + par

                @pl.when(k < n)
                def _():
                    pltpu.make_async_copy(
                        wt_hbm.at[:, pl.ds(0, 128)], ablk[par], gsem[par]
                    ).wait()

                    @pl.when(k >= 2)
                    def _():
                        pltpu.make_async_copy(
                            bblk[par], pairs_hbm.at[pl.ds(0, 64)], wsem[par]
                        ).wait()

                    @plsc.parallel_loop(0, D, unroll=2)
                    def _(q0):
                        permute(ablk[par], bblk[par], q0, 63)

                    out_dma(k, par)

                    @pl.when(k + 2 < n)
                    def _():
                        in_dma(k + 2, par)

            return carry

        lax.fori_loop(0, 13, body, 0)
        for par in range(2):

            @pl.when(n > par)
            def _():
                pltpu.make_async_copy(
                    bblk[par], pairs_hbm.at[pl.ds(0, 64)], wsem[par]
                ).wait()

        @pl.when(wid == NW - 1)
        def _():
            pltpu.sync_copy(wt_hbm.at[:, pl.ds(TAIL0, TAILW)], atail)

            @plsc.parallel_loop(0, TAILW // 2, unroll=2)
            def _(q0):
                permute(atail, btail, q0, TAILW // 2 - 1)

            pltpu.sync_copy(btail, pairs_hbm.at[pl.ds(P - TAILW // 2, TAILW // 2)])

    return transpose_kernel


@functools.cache
def _kb():
    mesh = plsc.VectorSubcoreMesh(core_axis_name="c", subcore_axis_name="s")

    @functools.partial(
        pl.kernel,
        mesh=mesh,
        out_type=jax.ShapeDtypeStruct((D, V), jnp.float32),
        scratch_types=[
            pltpu.VMEM((3200,), jnp.int32),
            pltpu.VMEM((128,), jnp.int32),
            pltpu.VMEM((128,), jnp.int32),
            pltpu.VMEM((128,), jnp.int32),
            pltpu.VMEM((128, 128), jnp.float32),
            pltpu.VMEM((128, 128), jnp.float32),
            pltpu.VMEM((128, 128), jnp.float32),
            pltpu.VMEM((D, 128), jnp.float32),
            pltpu.VMEM((D, 128), jnp.float32),
            pltpu.VMEM((TAILW,), jnp.int32),
            pltpu.VMEM((TAILW,), jnp.int32),
            pltpu.VMEM((TAILW, 128), jnp.float32),
            pltpu.VMEM((D, TAILW), jnp.float32),
            pltpu.SemaphoreType.DMA,
            pltpu.SemaphoreType.DMA,
            pltpu.SemaphoreType.DMA,
            pltpu.SemaphoreType.DMA,
            pltpu.SemaphoreType.DMA,
            pltpu.SemaphoreType.DMA,
        ],
        compiler_params=CP,
    )
    def gather_kernel(
        pairs_hbm,
        idx_hbm,
        out_hbm,
        cidx,
        p0,
        p1,
        p2,
        gb0,
        gb1,
        gb2,
        ob0,
        ob1,
        cidxt,
        pidxt,
        gtail,
        otail,
        g0,
        g1,
        g2,
        w0,
        w1,
        tsem,
    ):
        wid = lax.axis_index("s") * NC + lax.axis_index("c")
        start, n = _worker_range(wid)
        bufst = jnp.minimum(start, NB - 25)
        lo = start - bufst
        pidx = (p0, p1, p2)
        gblk = (gb0, gb1, gb2)
        oblk = (ob0, ob1)
        gsem = (g0, g1, g2)
        wsem = (w0, w1)
        io = _iota16()

        pltpu.sync_copy(idx_hbm.at[pl.ds(bufst * 128, 3200)], cidx)

        def compute_pidx_and_fire(k, par):
            base = (lo + k) * 128
            for g in range(8):
                pidx[par][pl.ds(16 * g, 16)] = (
                    cidx[pl.ds(base + 16 * g, 16)] >> 1
                )
            pltpu.async_copy(pairs_hbm.at[pidx[par]], gblk[par], gsem[par])

        compute_pidx_and_fire(0, 0)

        @pl.when(n > 1)
        def _():
            compute_pidx_and_fire(1, 1)

        @pl.when(n > 2)
        def _():
            compute_pidx_and_fire(2, 2)

        def body(i, carry):
            for j in range(6):
                k = 6 * i + j
                p3 = j % 3
                pr = j % 2

                @pl.when(k < n)
                def _():
                    pltpu.make_async_copy(
                        pairs_hbm.at[pidx[p3]], gblk[p3], gsem[p3]
                    ).wait()

                    @pl.when(k >= 2)
                    def _():
                        pltpu.make_async_copy(
                            oblk[pr], out_hbm.at[:, pl.ds(0, 128)], wsem[pr]
                        ).wait()

                    base = (lo + k) * 128
                    h64 = tuple(
                        (cidx[pl.ds(base + 16 * g, 16)] & 1) << 6
                        for g in range(8)
                    )

                    @plsc.parallel_loop(0, D, unroll=2)
                    def _(f0):
                        fv = (f0 + io) & 63
                        for g in range(8):
                            vals = plsc.load_gather(
                                gblk[p3], [16 * g + io, h64[g] + fv]
                            )
                            plsc.store_scatter(
                                oblk[pr], [fv, 16 * g + io], vals
                            )
                    pltpu.async_copy(
                        oblk[pr],
                        out_hbm.at[:, pl.ds((start + k) * 128, 128)],
                        wsem[pr],
                    )

                    @pl.when(k + 3 < n)
                    def _():
                        compute_pidx_and_fire(k + 3, p3)

            return carry

        lax.fori_loop(0, 5, body, 0)
        for par in range(2):

            @pl.when(n > par)
            def _():
                pltpu.make_async_copy(
                    oblk[par], out_hbm.at[:, pl.ds(0, 128)], wsem[par]
                ).wait()

        @pl.when(wid == NW - 1)
        def _():
            pltpu.sync_copy(idx_hbm.at[pl.ds(TAIL0, TAILW)], cidxt)
            for g in range(2):
                pidxt[pl.ds(16 * g, 16)] = cidxt[pl.ds(16 * g, 16)] >> 1
            pltpu.async_copy(pairs_hbm.at[pidxt], gtail, tsem).wait()
            h64t = tuple(
                (cidxt[pl.ds(16 * g, 16)] & 1) << 6 for g in range(2)
            )

            @plsc.parallel_loop(0, D, unroll=2)
            def _(f0):
                fv = (f0 + io) & 63
                for g in range(2):
                    vals = plsc.load_gather(
                        gtail, [16 * g + io, h64t[g] + fv]
                    )
                    plsc.store_scatter(otail, [fv, 16 * g + io], vals)
            pltpu.sync_copy(otail, out_hbm.at[:, pl.ds(TAIL0, TAILW)])

    return gather_kernel


def kernel(x, emb_weight):
    idx = x[:, 0].astype(jnp.int32)
    wt = emb_weight.T
    pairs = _ka()(wt)
    out_t = _kb()(pairs, idx)
    return out_t.T
